# baseline (speedup 1.0000x reference)
import functools
import math

import jax
import jax.numpy as jnp
from jax import lax
from jax.experimental import pallas as pl
from jax.experimental.pallas import tpu as pltpu
from jax.experimental.pallas import tpu_sc as plsc

_NC = 2
_NS = 16
_NW = _NC * _NS
_CH = 128
_RING = 4


def _emb_body(scale, n_chunks, table_hbm, idx_hbm, out_hbm,
              tbl_sp, tbuf_v, idx_v, rows0, rows1, rows2, rows3,
              g0, g1, g2, g3, o0, o1, o2, o3):
    c = lax.axis_index("c")
    s = lax.axis_index("s")
    rows_v = (rows0, rows1, rows2, rows3)
    gsem = (g0, g1, g2, g3)
    osem = (o0, o1, o2, o3)
    d = tbl_sp.shape[1]

    wid = s * _NC + c
    chunk0 = wid * n_chunks
    pltpu.async_copy(idx_hbm.at[pl.ds(chunk0, n_chunks)], idx_v, g0)

    rows_per_sub = tbl_sp.shape[0] // _NS
    base_r = s * rows_per_sub
    pltpu.sync_copy(table_hbm.at[pl.ds(base_r, rows_per_sub)], tbuf_v)
    vecs_per_row = d // 16

    def mul_body(r, carry):
        for cv in range(vecs_per_row):
            col = cv * 16
            tbuf_v[r, pl.ds(col, 16)] = (
                tbuf_v[r, pl.ds(col, 16)] * jnp.float32(scale))
        return carry

    lax.fori_loop(0, rows_per_sub, mul_body, 0)
    pltpu.sync_copy(tbuf_v, tbl_sp.at[pl.ds(base_r, rows_per_sub)])
    pltpu.make_async_copy(idx_hbm.at[pl.ds(chunk0, n_chunks)], idx_v, g0).wait()
    plsc.subcore_barrier()


    def start_gather(k, j):
        pltpu.async_copy(tbl_sp.at[idx_v.at[k]], rows_v[j], gsem[j])

    def wait_gather(k, j):
        pltpu.make_async_copy(tbl_sp.at[idx_v.at[k]], rows_v[j], gsem[j]).wait()

    def start_store(k, j):
        pltpu.async_copy(
            rows_v[j], out_hbm.at[pl.ds((chunk0 + k) * _CH, _CH)], osem[j])

    def wait_store(k, j):
        pltpu.make_async_copy(
            rows_v[j], out_hbm.at[pl.ds((chunk0 + k) * _CH, _CH)], osem[j]).wait()

    for j in range(_RING):
        start_gather(j, j)

    n_outer = n_chunks // _RING

    def ring_body(p, carry):
        kbase = p * _RING
        for j in range(_RING):
            wait_gather(kbase + j, j)
            start_store(kbase + j, j)

        @pl.when(p < n_outer - 1)
        def _():
            for j in range(_RING):
                wait_store(kbase + j, j)
                start_gather(kbase + _RING + j, j)

        return carry

    lax.fori_loop(0, n_outer, ring_body, 0)
    for j in range(_RING):
        wait_store(n_chunks - _RING + j, j)


def kernel(x, table):
    vocab, d = table.shape
    scale = math.sqrt(float(d))
    b_total = x.shape[0] * x.shape[1]
    b_per_w = b_total // _NW
    n_chunks = b_per_w // _CH
    vocab_pad = ((vocab + 63) // 64) * 64
    table_p = jnp.pad(table, ((0, vocab_pad - vocab), (0, 0)))
    idx = x.reshape(b_total // _CH, _CH).astype(jnp.int32)

    mesh = plsc.VectorSubcoreMesh(core_axis_name="c", subcore_axis_name="s")
    run = pl.kernel(
        functools.partial(_emb_body, scale, n_chunks),
        mesh=mesh,
        out_type=jax.ShapeDtypeStruct((b_total, d), jnp.float32),
        scratch_types=[
            pltpu.VMEM_SHARED((vocab_pad, d), jnp.float32),
            pltpu.VMEM((vocab_pad // _NS, d), jnp.float32),
            pltpu.VMEM((n_chunks, _CH), jnp.int32),
        ] + [pltpu.VMEM((_CH, d), jnp.float32) for _ in range(_RING)]
          + [pltpu.SemaphoreType.DMA for _ in range(2 * _RING)],
    )
    out = run(table_p, idx)
    return out.reshape(x.shape[0], x.shape[1], d)


# speedup vs baseline: 1.0011x; 1.0011x over previous
import functools
import math

import jax
import jax.numpy as jnp
from jax import lax
from jax.experimental import pallas as pl
from jax.experimental.pallas import tpu as pltpu
from jax.experimental.pallas import tpu_sc as plsc

_NC = 2
_NS = 16
_NW = _NC * _NS
_CH = 128
_PAIR = 2
_NP = 2


def _emb_body(scale, n_chunks, table_hbm, idx_hbm, out_hbm,
              tbl_sp, tbuf_v, idx_v, rowsA, rowsB,
              gA0, gA1, gB0, gB1, oA, oB):
    c = lax.axis_index("c")
    s = lax.axis_index("s")
    rows_v = (rowsA, rowsB)
    gsem = ((gA0, gA1), (gB0, gB1))
    osem = (oA, oB)
    d = tbl_sp.shape[1]

    wid = s * _NC + c
    chunk0 = wid * n_chunks
    pltpu.async_copy(idx_hbm.at[pl.ds(chunk0, n_chunks)], idx_v, gA0)

    rows_per_sub = tbl_sp.shape[0] // _NS
    base_r = s * rows_per_sub
    pltpu.sync_copy(table_hbm.at[pl.ds(base_r, rows_per_sub)], tbuf_v)
    vecs_per_row = d // 16

    def mul_body(r, carry):
        for cv in range(vecs_per_row):
            col = cv * 16
            tbuf_v[r, pl.ds(col, 16)] = (
                tbuf_v[r, pl.ds(col, 16)] * jnp.float32(scale))
        return carry

    lax.fori_loop(0, rows_per_sub, mul_body, 0)
    pltpu.sync_copy(tbuf_v, tbl_sp.at[pl.ds(base_r, rows_per_sub)])
    pltpu.make_async_copy(idx_hbm.at[pl.ds(chunk0, n_chunks)], idx_v, gA0).wait()
    plsc.subcore_barrier()

    def start_gather(k, p, h):
        pltpu.async_copy(tbl_sp.at[idx_v.at[k]],
                         rows_v[p].at[pl.ds(h * _CH, _CH)], gsem[p][h])

    def wait_gather(k, p, h):
        pltpu.make_async_copy(tbl_sp.at[idx_v.at[k]],
                              rows_v[p].at[pl.ds(h * _CH, _CH)],
                              gsem[p][h]).wait()

    def start_store(k, p):
        pltpu.async_copy(
            rows_v[p], out_hbm.at[pl.ds((chunk0 + k) * _CH, _PAIR * _CH)],
            osem[p])

    def wait_store(k, p):
        pltpu.make_async_copy(
            rows_v[p], out_hbm.at[pl.ds((chunk0 + k) * _CH, _PAIR * _CH)],
            osem[p]).wait()

    for p in range(_NP):
        for h in range(_PAIR):
            start_gather(p * _PAIR + h, p, h)

    n_outer = n_chunks // (_NP * _PAIR)

    def ring_body(q, carry):
        kbase = q * _NP * _PAIR
        for p in range(_NP):
            for h in range(_PAIR):
                wait_gather(kbase + p * _PAIR + h, p, h)
            start_store(kbase + p * _PAIR, p)

        @pl.when(q < n_outer - 1)
        def _():
            for p in range(_NP):
                wait_store(kbase + p * _PAIR, p)
                for h in range(_PAIR):
                    start_gather(kbase + _NP * _PAIR + p * _PAIR + h, p, h)

        return carry

    lax.fori_loop(0, n_outer, ring_body, 0)
    for p in range(_NP):
        wait_store(n_chunks - _NP * _PAIR + p * _PAIR, p)


def kernel(x, table):
    vocab, d = table.shape
    scale = math.sqrt(float(d))
    b_total = x.shape[0] * x.shape[1]
    b_per_w = b_total // _NW
    n_chunks = b_per_w // _CH
    vocab_pad = ((vocab + 63) // 64) * 64
    table_p = jnp.pad(table, ((0, vocab_pad - vocab), (0, 0)))
    idx = x.reshape(b_total // _CH, _CH).astype(jnp.int32)

    mesh = plsc.VectorSubcoreMesh(core_axis_name="c", subcore_axis_name="s")
    run = pl.kernel(
        functools.partial(_emb_body, scale, n_chunks),
        mesh=mesh,
        out_type=jax.ShapeDtypeStruct((b_total, d), jnp.float32),
        scratch_types=[
            pltpu.VMEM_SHARED((vocab_pad, d), jnp.float32),
            pltpu.VMEM((vocab_pad // _NS, d), jnp.float32),
            pltpu.VMEM((n_chunks, _CH), jnp.int32),
            pltpu.VMEM((_PAIR * _CH, d), jnp.float32),
            pltpu.VMEM((_PAIR * _CH, d), jnp.float32),
        ] + [pltpu.SemaphoreType.DMA for _ in range(6)],
    )
    out = run(table_p, idx)
    return out.reshape(x.shape[0], x.shape[1], d)


# speedup vs baseline: 1.0154x; 1.0143x over previous
import functools
import math

import jax
import jax.numpy as jnp
from jax import lax
from jax.experimental import pallas as pl
from jax.experimental.pallas import tpu as pltpu
from jax.experimental.pallas import tpu_sc as plsc

_NC = 2
_NS = 16
_NW = _NC * _NS
_CA = 128
_NP = 2


def _emb_body(scale, hist, rows_per_w, table_hbm, idx_hbm, out_hbm,
              tbl_sp, tbuf_v, idx_v, rowsA, rowsB,
              gA0, gA1, gB0, gB1, oA, oB):
    c = lax.axis_index("c")
    s = lax.axis_index("s")
    rows_v = (rowsA, rowsB)
    gsem = ((gA0, gA1), (gB0, gB1))
    osem = (oA, oB)
    d = tbl_sp.shape[1]
    cb = hist - _CA

    wid = s * _NC + c
    row0 = wid * rows_per_w
    pltpu.async_copy(idx_hbm.at[pl.ds(row0, rows_per_w)], idx_v, gA0)

    rows_per_sub = tbl_sp.shape[0] // _NS
    base_r = s * rows_per_sub
    pltpu.sync_copy(table_hbm.at[pl.ds(base_r, rows_per_sub)], tbuf_v)
    vecs_per_row = d // 16

    def mul_body(r, carry):
        for cv in range(vecs_per_row):
            col = cv * 16
            tbuf_v[r, pl.ds(col, 16)] = (
                tbuf_v[r, pl.ds(col, 16)] * jnp.float32(scale))
        return carry

    lax.fori_loop(0, rows_per_sub, mul_body, 0)
    pltpu.sync_copy(tbuf_v, tbl_sp.at[pl.ds(base_r, rows_per_sub)])
    pltpu.make_async_copy(idx_hbm.at[pl.ds(row0, rows_per_w)], idx_v, gA0).wait()
    plsc.subcore_barrier()

    def start_gathers(r, p):
        pltpu.async_copy(tbl_sp.at[idx_v.at[r, pl.ds(0, _CA)]],
                         rows_v[p].at[pl.ds(0, _CA)], gsem[p][0])
        pltpu.async_copy(tbl_sp.at[idx_v.at[r, pl.ds(_CA, cb)]],
                         rows_v[p].at[pl.ds(_CA, cb)], gsem[p][1])

    def wait_gathers(r, p):
        pltpu.make_async_copy(tbl_sp.at[idx_v.at[r, pl.ds(0, _CA)]],
                              rows_v[p].at[pl.ds(0, _CA)], gsem[p][0]).wait()
        pltpu.make_async_copy(tbl_sp.at[idx_v.at[r, pl.ds(_CA, cb)]],
                              rows_v[p].at[pl.ds(_CA, cb)], gsem[p][1]).wait()

    def start_store(r, p):
        pltpu.async_copy(
            rows_v[p], out_hbm.at[pl.ds((row0 + r) * hist, hist)], osem[p])

    def wait_store(r, p):
        pltpu.make_async_copy(
            rows_v[p], out_hbm.at[pl.ds((row0 + r) * hist, hist)],
            osem[p]).wait()

    for p in range(_NP):
        start_gathers(p, p)

    n_outer = rows_per_w // _NP

    def ring_body(q, carry):
        rbase = q * _NP
        for p in range(_NP):
            wait_gathers(rbase + p, p)
            start_store(rbase + p, p)

        @pl.when(q < n_outer - 1)
        def _():
            for p in range(_NP):
                wait_store(rbase + p, p)
                start_gathers(rbase + _NP + p, p)

        return carry

    lax.fori_loop(0, n_outer, ring_body, 0)
    for p in range(_NP):
        wait_store(rows_per_w - _NP + p, p)


def kernel(x, table):
    vocab, d = table.shape
    batch, hist = x.shape
    scale = math.sqrt(float(d))
    b_total = batch * hist
    rows_per_w = batch // _NW
    vocab_pad = ((vocab + 63) // 64) * 64
    table_p = jnp.pad(table, ((0, vocab_pad - vocab), (0, 0)))
    idx = x.astype(jnp.int32)

    mesh = plsc.VectorSubcoreMesh(core_axis_name="c", subcore_axis_name="s")
    run = pl.kernel(
        functools.partial(_emb_body, scale, hist, rows_per_w),
        mesh=mesh,
        out_type=jax.ShapeDtypeStruct((b_total, d), jnp.float32),
        scratch_types=[
            pltpu.VMEM_SHARED((vocab_pad, d), jnp.float32),
            pltpu.VMEM((vocab_pad // _NS, d), jnp.float32),
            pltpu.VMEM((rows_per_w, hist), jnp.int32),
            pltpu.VMEM((hist, d), jnp.float32),
            pltpu.VMEM((hist, d), jnp.float32),
        ] + [pltpu.SemaphoreType.DMA for _ in range(6)],
    )
    out = run(table_p, idx)
    return out.reshape(batch, hist, d)
